# Initial kernel scaffold; baseline (speedup 1.0000x reference)
import jax
import jax.numpy as jnp
from jax.experimental import pallas as pl


def kernel(user_ids, item_ids, user_table, item_table):
    raise NotImplementedError("write your pallas kernel here")


# speedup vs baseline: 4.1836x; 4.1836x over previous
import functools

import jax
import jax.numpy as jnp
from jax import lax
from jax.experimental import pallas as pl
from jax.experimental.pallas import tpu as pltpu
from jax.experimental.pallas import tpu_sc as plsc


def _build(B, S, D, V):
    NW = 32
    item_total = B * S
    item_per_w = item_total // NW
    user_per_w = B // NW
    CH = 1024
    n_chunks = item_per_w // CH

    mesh = plsc.VectorSubcoreMesh(core_axis_name="c", subcore_axis_name="s")

    @functools.partial(
        pl.kernel,
        mesh=mesh,
        out_type=[
            jax.ShapeDtypeStruct((B, D), jnp.float32),
            jax.ShapeDtypeStruct((item_total, D), jnp.float32),
        ],
        scratch_types=[
            pltpu.VMEM((user_per_w,), jnp.int32),
            pltpu.VMEM((user_per_w, D), jnp.float32),
            pltpu.VMEM((CH,), jnp.int32),
            pltpu.VMEM((CH, D), jnp.float32),
            pltpu.SemaphoreType.DMA,
        ],
        compiler_params=pltpu.CompilerParams(use_tc_tiling_on_sc=False),
    )
    def k(uids, iids, utab, itab, uout, iout, uidx_v, urows_v, idx_v, rows_v, sem):
        wid = lax.axis_index("s") * 2 + lax.axis_index("c")

        ubase = wid * user_per_w
        pltpu.sync_copy(uids.at[pl.ds(ubase, user_per_w)], uidx_v)
        pltpu.async_copy(utab.at[uidx_v], urows_v, sem).wait()
        pltpu.sync_copy(urows_v, uout.at[pl.ds(ubase, user_per_w)])

        ibase = wid * item_per_w

        def body(i, carry):
            off = ibase + i * CH
            pltpu.sync_copy(iids.at[pl.ds(off, CH)], idx_v)
            pltpu.async_copy(itab.at[idx_v], rows_v, sem).wait()
            pltpu.sync_copy(rows_v, iout.at[pl.ds(off, CH)])
            return carry

        lax.fori_loop(0, n_chunks, body, 0)

    return k


def kernel(user_ids, item_ids, user_table, item_table):
    B, S = item_ids.shape
    V, D = user_table.shape
    k = _build(B, S, D, V)
    uout, iout = k(user_ids, item_ids.reshape(-1), user_table, item_table)
    return uout, iout.reshape(B, S, D)


# speedup vs baseline: 4.3501x; 1.0398x over previous
import functools

import jax
import jax.numpy as jnp
from jax import lax
from jax.experimental import pallas as pl
from jax.experimental.pallas import tpu as pltpu
from jax.experimental.pallas import tpu_sc as plsc


def _build(B, S, D, V):
    NW = 32
    item_total = B * S
    item_per_w = item_total // NW
    user_per_w = B // NW
    CH = 1600
    n_chunks = item_per_w // CH

    mesh = plsc.VectorSubcoreMesh(core_axis_name="c", subcore_axis_name="s")

    @functools.partial(
        pl.kernel,
        mesh=mesh,
        out_type=[
            jax.ShapeDtypeStruct((B, D), jnp.float32),
            jax.ShapeDtypeStruct((item_total, D), jnp.float32),
        ],
        scratch_types=[
            pltpu.VMEM((user_per_w,), jnp.int32),
            pltpu.VMEM((user_per_w, D), jnp.float32),
            pltpu.VMEM((CH,), jnp.int32),
            pltpu.VMEM((CH,), jnp.int32),
            pltpu.VMEM((CH, D), jnp.float32),
            pltpu.VMEM((CH, D), jnp.float32),
            pltpu.SemaphoreType.DMA,
            pltpu.SemaphoreType.DMA,
            pltpu.SemaphoreType.DMA,
            pltpu.SemaphoreType.DMA,
            pltpu.SemaphoreType.DMA,
            pltpu.SemaphoreType.DMA,
        ],
        compiler_params=pltpu.CompilerParams(use_tc_tiling_on_sc=False),
    )
    def k(uids, iids, utab, itab, uout, iout,
          uidx_v, urows_v, idx0, idx1, rows0, rows1,
          sl0, sl1, sg0, sg1, sw0, sw1):
        wid = lax.axis_index("s") * 2 + lax.axis_index("c")

        ubase = wid * user_per_w
        pltpu.sync_copy(uids.at[pl.ds(ubase, user_per_w)], uidx_v)
        pltpu.async_copy(utab.at[uidx_v], urows_v, sg0).wait()
        pltpu.sync_copy(urows_v, uout.at[pl.ds(ubase, user_per_w)])

        ibase = wid * item_per_w

        def off(c):
            return ibase + jnp.minimum(c, n_chunks - 1) * CH

        def load(c, idx_v, sem):
            return pltpu.async_copy(iids.at[pl.ds(off(c), CH)], idx_v, sem)

        load(0, idx0, sl0).wait()
        load(1, idx1, sl1)
        g0 = pltpu.async_copy(itab.at[idx0], rows0, sg0)

        def body(j, carry):
            c = 2 * j
            pltpu.make_async_copy(itab.at[idx0], rows0, sg0).wait()
            pltpu.async_copy(rows0, iout.at[pl.ds(off(c), CH)], sw0)
            pltpu.make_async_copy(iids.at[pl.ds(off(c + 1), CH)], idx1, sl1).wait()
            pltpu.async_copy(itab.at[idx1], rows1, sg1)
            load(c + 2, idx0, sl0)
            pltpu.make_async_copy(itab.at[idx1], rows1, sg1).wait()
            pltpu.async_copy(rows1, iout.at[pl.ds(off(c + 1), CH)], sw1)
            pltpu.make_async_copy(iids.at[pl.ds(off(c + 2), CH)], idx0, sl0).wait()
            pltpu.make_async_copy(rows0, iout.at[pl.ds(off(c), CH)], sw0).wait()
            pltpu.async_copy(itab.at[idx0], rows0, sg0)
            load(c + 3, idx1, sl1)
            pltpu.make_async_copy(rows1, iout.at[pl.ds(off(c + 1), CH)], sw1).wait()
            return carry

        lax.fori_loop(0, n_chunks // 2, body, 0)

        pltpu.make_async_copy(itab.at[idx0], rows0, sg0).wait()
        pltpu.make_async_copy(iids.at[pl.ds(0, CH)], idx1, sl1).wait()

    return k


def kernel(user_ids, item_ids, user_table, item_table):
    B, S = item_ids.shape
    V, D = user_table.shape
    k = _build(B, S, D, V)
    uout, iout = k(user_ids, item_ids.reshape(-1), user_table, item_table)
    return uout, iout.reshape(B, S, D)


# speedup vs baseline: 4.7053x; 1.0816x over previous
import functools

import jax
import jax.numpy as jnp
from jax import lax
from jax.experimental import pallas as pl
from jax.experimental.pallas import tpu as pltpu
from jax.experimental.pallas import tpu_sc as plsc


def _build(B, S, D, V):
    NW = 32
    CH = 1024
    CPS = B // CH
    n_chunks = S * CPS
    chunks_per_w = n_chunks // NW
    user_per_w = B // NW

    mesh = plsc.VectorSubcoreMesh(core_axis_name="c", subcore_axis_name="s")

    @functools.partial(
        pl.kernel,
        mesh=mesh,
        out_type=[
            jax.ShapeDtypeStruct((B, D), jnp.float32),
            jax.ShapeDtypeStruct((S, B, D), jnp.float32),
        ],
        scratch_types=[
            pltpu.VMEM((user_per_w,), jnp.int32),
            pltpu.VMEM((user_per_w, D), jnp.float32),
            pltpu.VMEM((CH,), jnp.int32),
            pltpu.VMEM((CH,), jnp.int32),
            pltpu.VMEM((CH, D), jnp.float32),
            pltpu.VMEM((CH, D), jnp.float32),
            pltpu.SemaphoreType.DMA,
            pltpu.SemaphoreType.DMA,
            pltpu.SemaphoreType.DMA,
            pltpu.SemaphoreType.DMA,
            pltpu.SemaphoreType.DMA,
            pltpu.SemaphoreType.DMA,
            pltpu.SemaphoreType.DMA,
        ],
        compiler_params=pltpu.CompilerParams(use_tc_tiling_on_sc=False),
    )
    def k(uids, iids_t, utab, itab, uout, iout_t,
          uidx_v, urows_v, idx0, idx1, rows0, rows1,
          usem, sl0, sl1, sg0, sg1, sw0, sw1):
        wid = lax.axis_index("s") * 2 + lax.axis_index("c")

        ubase = wid * user_per_w
        pltpu.sync_copy(uids.at[pl.ds(ubase, user_per_w)], uidx_v)
        pltpu.async_copy(utab.at[uidx_v], urows_v, usem).wait()
        pltpu.sync_copy(urows_v, uout.at[pl.ds(ubase, user_per_w)])

        c0 = wid * chunks_per_w
        last = n_chunks - 1

        def parts(c):
            cc = jnp.minimum(c, last)
            return cc // CPS, (cc % CPS) * CH

        def load(c, idx_v, sem):
            s, b0 = parts(c)
            return pltpu.make_async_copy(iids_t.at[s, pl.ds(b0, CH)], idx_v, sem)

        def gath(idx_v, rows_v, sem):
            return pltpu.make_async_copy(itab.at[idx_v], rows_v, sem)

        def wr(c, rows_v, sem):
            s, b0 = parts(c)
            return pltpu.make_async_copy(rows_v, iout_t.at[s, pl.ds(b0, CH)], sem)

        load(c0, idx0, sl0).start()
        load(c0, idx0, sl0).wait()
        load(c0 + 1, idx1, sl1).start()
        gath(idx0, rows0, sg0).start()

        def body(j, carry):
            c = c0 + 2 * j
            gath(idx0, rows0, sg0).wait()
            wr(c, rows0, sw0).start()
            load(c + 1, idx1, sl1).wait()
            gath(idx1, rows1, sg1).start()
            load(c + 2, idx0, sl0).start()
            gath(idx1, rows1, sg1).wait()
            wr(c + 1, rows1, sw1).start()
            load(c + 2, idx0, sl0).wait()
            wr(c, rows0, sw0).wait()
            gath(idx0, rows0, sg0).start()
            load(c + 3, idx1, sl1).start()
            wr(c + 1, rows1, sw1).wait()
            return carry

        lax.fori_loop(0, chunks_per_w // 2, body, 0)

        gath(idx0, rows0, sg0).wait()
        load(c0, idx1, sl1).wait()

    return k


def kernel(user_ids, item_ids, user_table, item_table):
    B, S = item_ids.shape
    V, D = user_table.shape
    k = _build(B, S, D, V)
    uout, iout_t = k(user_ids, item_ids.T, user_table, item_table)
    return uout, iout_t.transpose(1, 0, 2)


# speedup vs baseline: 9.0333x; 1.9198x over previous
import functools

import jax
import jax.numpy as jnp
from jax import lax
from jax.experimental import pallas as pl
from jax.experimental.pallas import tpu as pltpu
from jax.experimental.pallas import tpu_sc as plsc


def _build(B, S, D, V):
    NW = 32
    CH = 512
    TPC = CH // 128
    CPS = B // CH
    SB = B // 128
    n_chunks = S * CPS
    chunks_per_w = n_chunks // NW
    user_per_w = B // NW
    JT = D // 8

    mesh = plsc.VectorSubcoreMesh(core_axis_name="c", subcore_axis_name="s")

    @functools.partial(
        pl.kernel,
        mesh=mesh,
        out_type=[
            jax.ShapeDtypeStruct((B, D), jnp.float32),
            jax.ShapeDtypeStruct((S, JT, SB, 8, 128), jnp.float32),
        ],
        scratch_types=[
            pltpu.VMEM((user_per_w,), jnp.int32),
            pltpu.VMEM((user_per_w, D), jnp.float32),
            pltpu.VMEM((CH,), jnp.int32),
            pltpu.VMEM((CH,), jnp.int32),
            pltpu.VMEM((CH, D), jnp.float32),
            pltpu.VMEM((CH, D), jnp.float32),
            pltpu.VMEM((JT, TPC, 8, 128), jnp.float32),
            pltpu.VMEM((JT, TPC, 8, 128), jnp.float32),
            pltpu.SemaphoreType.DMA,
            pltpu.SemaphoreType.DMA,
            pltpu.SemaphoreType.DMA,
            pltpu.SemaphoreType.DMA,
            pltpu.SemaphoreType.DMA,
            pltpu.SemaphoreType.DMA,
            pltpu.SemaphoreType.DMA,
        ],
        compiler_params=pltpu.CompilerParams(use_tc_tiling_on_sc=False),
    )
    def k(uids, iids_t, utab, itab, uout, iout5,
          uidx_v, urows_v, idx0, idx1, rows0, rows1, out0, out1,
          usem, sl0, sl1, sg0, sg1, sw0, sw1):
        wid = lax.axis_index("s") * 2 + lax.axis_index("c")

        ubase = wid * user_per_w
        pltpu.sync_copy(uids.at[pl.ds(ubase, user_per_w)], uidx_v)
        pltpu.async_copy(utab.at[uidx_v], urows_v, usem).wait()
        pltpu.sync_copy(urows_v, uout.at[pl.ds(ubase, user_per_w)])

        c0 = wid * chunks_per_w
        last = n_chunks - 1

        def parts(c):
            cc = jnp.minimum(c, last)
            return cc // CPS, (cc % CPS) * TPC

        def load(c, idx_v, sem):
            s, btl0 = parts(c)
            return pltpu.make_async_copy(
                iids_t.at[s, pl.ds(btl0 * 128, CH)], idx_v, sem)

        def gath(idx_v, rows_v, sem):
            return pltpu.make_async_copy(itab.at[idx_v], rows_v, sem)

        def wr(c, out_v, sem, wait):
            s, btl0 = parts(c)
            for jt in range(JT):
                cp = pltpu.make_async_copy(
                    out_v.at[jt], iout5.at[s, jt, pl.ds(btl0, TPC)], sem)
                if wait:
                    cp.wait()
                else:
                    cp.start()

        iota = lax.iota(jnp.int32, 16)
        masks = {d: (iota & d) == 0 for d in (8, 4, 2, 1)}
        perms = {d: iota ^ d for d in (8, 4, 2, 1)}

        def transpose_chunk(rows_v, out_v):
            for btl in range(TPC):
                def tbody(t, carry):
                    bb = btl * 128 + t * 16
                    for jg in range(D // 16):
                        a = [rows_v[bb + r, pl.ds(jg * 16, 16)]
                             for r in range(16)]
                        for d in (8, 4, 2, 1):
                            for i in range(16):
                                if i & d:
                                    continue
                                A, B = a[i], a[i + d]
                                Ax = A[perms[d]]
                                Bx = B[perms[d]]
                                a[i] = jnp.where(masks[d], A, Bx)
                                a[i + d] = jnp.where(masks[d], Ax, B)
                        for jj in range(16):
                            jgl = jg * 16 + jj
                            out_v[jgl // 8, btl, jgl % 8,
                                  pl.ds(t * 16, 16)] = a[jj]
                    return carry
                lax.fori_loop(0, 8, tbody, 0)

        load(c0, idx0, sl0).start()
        load(c0, idx0, sl0).wait()
        gath(idx0, rows0, sg0).start()
        load(c0 + 1, idx1, sl1).start()
        load(c0 + 1, idx1, sl1).wait()
        gath(idx1, rows1, sg1).start()
        wr(c0, out0, sw0, False)
        wr(c0 + 1, out1, sw1, False)

        def body(j, carry):
            c = c0 + 2 * j
            gath(idx0, rows0, sg0).wait()
            load(c + 2, idx0, sl0).start()
            wr(c, out0, sw0, True)
            transpose_chunk(rows0, out0)
            wr(c, out0, sw0, False)
            load(c + 2, idx0, sl0).wait()
            gath(idx0, rows0, sg0).start()
            gath(idx1, rows1, sg1).wait()
            load(c + 3, idx1, sl1).start()
            wr(c + 1, out1, sw1, True)
            transpose_chunk(rows1, out1)
            wr(c + 1, out1, sw1, False)
            load(c + 3, idx1, sl1).wait()
            gath(idx1, rows1, sg1).start()
            return carry

        lax.fori_loop(0, chunks_per_w // 2, body, 0)

        gath(idx0, rows0, sg0).wait()
        gath(idx1, rows1, sg1).wait()
        wr(c0 + chunks_per_w - 2, out0, sw0, True)
        wr(c0 + chunks_per_w - 1, out1, sw1, True)

    return k


def kernel(user_ids, item_ids, user_table, item_table):
    B, S = item_ids.shape
    V, D = user_table.shape
    k = _build(B, S, D, V)
    uout, iout5 = k(user_ids, item_ids.T, user_table, item_table)
    item_emb = iout5.transpose(2, 4, 0, 1, 3).reshape(B, S, D)
    return uout, item_emb
